# baseline (speedup 1.0000x reference)
import functools

import jax
import jax.numpy as jnp
from jax import lax
from jax.experimental import pallas as pl
from jax.experimental.pallas import tpu as pltpu
from jax.experimental.pallas import tpu_sc as plsc

_NC, _NS, _L = 2, 16, 16
_NW = _NC * _NS
_BS, _SEQ, _FEAT, _OUT = 8, 2048, 1024, 1024
_FPT = _FEAT // _NW
_HROWS = _SEQ // 2


def _sc_pool(feat_hbm, score_hbm, idx_hbm, zero_hbm, out_hbm, buf, idxb, scb, acc):
    c = lax.axis_index("c")
    s = lax.axis_index("s")
    w = s * _NC + c
    f0 = w * _FPT

    cols = [
        lax.iota(jnp.int32, _L) + k * _L for k in range(_FPT // _L)
    ]

    def _batch(b, carry):
        pltpu.sync_copy(zero_hbm, acc)
        pltpu.sync_copy(idx_hbm.at[b], idxb)
        pltpu.sync_copy(score_hbm.at[b], scb)

        for h in range(2):
            pltpu.sync_copy(
                feat_hbm.at[b, pl.ds(h * _HROWS, _HROWS), pl.ds(f0, _FPT)],
                buf,
            )

            def _grp(g, gi, h=h):
                iv = idxb[pl.ds(h * _HROWS + g * _L, _L)]
                sv = scb[pl.ds(h * _HROWS + g * _L, _L)]
                for r in range(_L):
                    row = g * _L + r
                    rowids = jnp.full((_L,), iv[r])
                    bc = jnp.full((_L,), sv[r])
                    for k in range(_FPT // _L):
                        v = buf[row, pl.ds(k * _L, _L)] * bc
                        plsc.addupdate_scatter(acc, [rowids, cols[k]], v)
                return gi

            lax.fori_loop(0, _HROWS // _L, _grp, 0)

        pltpu.sync_copy(acc, out_hbm.at[b, :, pl.ds(f0, _FPT)])
        return carry

    lax.fori_loop(0, _BS, _batch, 0)


_sc_call = functools.partial(
    pl.kernel,
    out_type=jax.ShapeDtypeStruct((_BS, _OUT, _FEAT), jnp.float32),
    mesh=plsc.VectorSubcoreMesh(core_axis_name="c", subcore_axis_name="s"),
    compiler_params=pltpu.CompilerParams(
        use_tc_tiling_on_sc=False, needs_layout_passes=False
    ),
    scratch_types=[
        pltpu.VMEM((_HROWS, _FPT), jnp.float32),
        pltpu.VMEM((_SEQ,), jnp.int32),
        pltpu.VMEM((_SEQ,), jnp.float32),
        pltpu.VMEM((_OUT, _FPT), jnp.float32),
    ],
)(_sc_pool)


def kernel(score, feature, out_len):
    s2 = score[:, :, 0]

    cumsum = jnp.cumsum(score, axis=1)
    cumsum = jnp.where(jnp.mod(cumsum, 1.0) < 0.01, cumsum - 0.01, cumsum)
    int_cumsum = jnp.floor(cumsum).astype(jnp.int32)
    int_cumsum = jnp.clip(int_cumsum, 0, out_len - 1)
    idx = int_cumsum[:, :, 0]

    zeros = jnp.zeros((_OUT, _FPT), jnp.float32)
    return _sc_call(feature, s2, idx, zeros)


# speedup vs baseline: 5.0413x; 5.0413x over previous
import jax
import jax.numpy as jnp
from jax.experimental import pallas as pl

_BS = 8
_SEQ = 2048
_FEAT = 1024
_OUT = 1024
_WIN = 256
_NW = _SEQ // _WIN
_SPAN = 272


def _wpool_kernel(idx_ref, score_ref, feat_ref, out_ref):
    w = pl.program_id(1)

    @pl.when(w == 0)
    def _():
        out_ref[...] = jnp.zeros_like(out_ref)

    v0 = idx_ref[0, 0, 0, 0]
    j0 = jnp.minimum((v0 // 8) * 8, _OUT - _SPAN)
    j0 = pl.multiple_of(j0, 8)
    rows = jax.lax.broadcasted_iota(jnp.int32, (_SPAN, 1), 0) + j0
    a = jnp.where(idx_ref[0, 0] == rows, score_ref[0, 0], 0.0)
    part = jax.lax.dot(a, feat_ref[0], preferred_element_type=jnp.float32)
    out_ref[0, pl.ds(j0, _SPAN), :] += part


def kernel(score, feature, out_len):
    s2 = score[:, :, 0]

    cumsum = jnp.cumsum(score, axis=1)
    cumsum = jnp.where(jnp.mod(cumsum, 1.0) < 0.01, cumsum - 0.01, cumsum)
    int_cumsum = jnp.floor(cumsum).astype(jnp.int32)
    int_cumsum = jnp.clip(int_cumsum, 0, out_len - 1)
    idx = int_cumsum[:, :, 0]

    idx4 = idx.reshape(_BS, _NW, 1, _WIN)
    s4 = s2.reshape(_BS, _NW, 1, _WIN)

    out = pl.pallas_call(
        _wpool_kernel,
        grid=(_BS, _NW),
        in_specs=[
            pl.BlockSpec((1, 1, 1, _WIN), lambda b, w: (b, w, 0, 0)),
            pl.BlockSpec((1, 1, 1, _WIN), lambda b, w: (b, w, 0, 0)),
            pl.BlockSpec((1, _WIN, _FEAT), lambda b, w: (b, w, 0)),
        ],
        out_specs=pl.BlockSpec((1, _OUT, _FEAT), lambda b, w: (b, 0, 0)),
        out_shape=jax.ShapeDtypeStruct((_BS, _OUT, _FEAT), jnp.float32),
    )(idx4, s4, feature)
    return out
